# Initial kernel scaffold; baseline (speedup 1.0000x reference)
import jax
import jax.numpy as jnp
from jax.experimental import pallas as pl


def kernel(x, emb, W1, b1, W2, b2):
    raise NotImplementedError("write your pallas kernel here")


# speedup vs baseline: 29.0633x; 29.0633x over previous
import functools

import jax
import jax.numpy as jnp
from jax import lax
from jax.experimental import pallas as pl
from jax.experimental.pallas import tpu as pltpu
from jax.experimental.pallas import tpu_sc as plsc

B = 4096
L = 200
D = 512
NCLS = 1000
VPAD = 1024

NW = 32
ROWS_PER_W = B // NW
CH = 64
NCHUNK = ROWS_PER_W // CH
NVEC = L // 16


def _sc_histogram(x):
    mesh = plsc.VectorSubcoreMesh(core_axis_name="c", subcore_axis_name="s")

    @functools.partial(
        pl.kernel,
        mesh=mesh,
        out_type=jax.ShapeDtypeStruct((B * VPAD,), jnp.float32),
        scratch_types=[
            pltpu.VMEM((CH * L,), jnp.int32),
            pltpu.VMEM((CH * VPAD,), jnp.float32),
        ],
        compiler_params=pltpu.CompilerParams(needs_layout_passes=False),
    )
    def hist_kernel(x_hbm, counts_hbm, idx_v, hist_v):
        wid = lax.axis_index("c") * 16 + lax.axis_index("s")
        ones = jnp.ones((16,), jnp.float32)
        zeros = jnp.zeros((16,), jnp.float32)
        lane = lax.iota(jnp.int32, 16)
        tail_mask = lane >= 8

        for c in range(NCHUNK):
            base = wid * ROWS_PER_W + c * CH
            pltpu.sync_copy(x_hbm.at[pl.ds(base * L, CH * L)], idx_v)

            def zero_blk(i, carry):
                hist_v[pl.ds(i * 16, 16)] = zeros
                return carry

            lax.fori_loop(0, CH * VPAD // 16, zero_blk, 0)

            def do_row(r, carry):
                rbase = r * VPAD
                for j in range(NVEC):
                    ids = idx_v[pl.ds(r * L + j * 16, 16)]
                    plsc.addupdate_scatter(hist_v, [ids + rbase], ones)
                ids = idx_v[pl.ds(r * L + L - 16, 16)]
                plsc.addupdate_scatter(hist_v, [ids + rbase], ones, mask=tail_mask)
                return carry

            lax.fori_loop(0, CH, do_row, 0)

            pltpu.sync_copy(hist_v, counts_hbm.at[pl.ds(base * VPAD, CH * VPAD)])

    return hist_kernel(x.reshape(B * L)).reshape(B, VPAD)


BB = 256


def _mlp_body(counts_ref, emb_ref, w1_ref, b1_ref, w2_ref, b2_ref, out_ref):
    c = counts_ref[...]
    denom = 200.0 - c[:, 0:1]
    pooled = jnp.dot(c, emb_ref[...], preferred_element_type=jnp.float32) / denom
    h = jnp.dot(pooled, w1_ref[...], preferred_element_type=jnp.float32) + b1_ref[...]
    h = jnp.maximum(h, 0.0)
    out_ref[...] = (
        jnp.dot(h, w2_ref[...], preferred_element_type=jnp.float32) + b2_ref[...]
    )


def _tc_mlp(counts, emb_z, W1, b1, W2, b2):
    return pl.pallas_call(
        _mlp_body,
        grid=(B // BB,),
        in_specs=[
            pl.BlockSpec((BB, VPAD), lambda i: (i, 0)),
            pl.BlockSpec((VPAD, D), lambda i: (0, 0)),
            pl.BlockSpec((D, D), lambda i: (0, 0)),
            pl.BlockSpec((1, D), lambda i: (0, 0)),
            pl.BlockSpec((D, NCLS), lambda i: (0, 0)),
            pl.BlockSpec((1, NCLS), lambda i: (0, 0)),
        ],
        out_specs=pl.BlockSpec((BB, NCLS), lambda i: (i, 0)),
        out_shape=jax.ShapeDtypeStruct((B, NCLS), jnp.float32),
    )(counts, emb_z, W1, b1.reshape(1, D), W2, b2.reshape(1, NCLS))


def kernel(x, emb, W1, b1, W2, b2):
    counts = _sc_histogram(x.astype(jnp.int32))
    emb_z = jnp.zeros((VPAD, D), emb.dtype).at[1:NCLS].set(emb[1:])
    return _tc_mlp(counts, emb_z, W1, b1, W2, b2)


# speedup vs baseline: 36.2322x; 1.2467x over previous
import functools

import jax
import jax.numpy as jnp
from jax import lax
from jax.experimental import pallas as pl
from jax.experimental.pallas import tpu as pltpu
from jax.experimental.pallas import tpu_sc as plsc

B = 4096
L = 200
D = 512
NCLS = 1000
VPAD = 1024

NW = 32
ROWS_PER_W = B // NW
CH = 64
NCHUNK = ROWS_PER_W // CH
NVEC = L // 16


def _sc_histogram(x):
    mesh = plsc.VectorSubcoreMesh(core_axis_name="c", subcore_axis_name="s")

    @functools.partial(
        pl.kernel,
        mesh=mesh,
        out_type=jax.ShapeDtypeStruct((B * VPAD,), jnp.float32),
        scratch_types=[
            pltpu.VMEM((CH * L,), jnp.int32),
            pltpu.VMEM((CH * VPAD,), jnp.float32),
        ],
        compiler_params=pltpu.CompilerParams(needs_layout_passes=False),
    )
    def hist_kernel(x_hbm, counts_hbm, idx_v, hist_v):
        wid = lax.axis_index("c") * 16 + lax.axis_index("s")
        ones = jnp.ones((16,), jnp.float32)
        zeros = jnp.zeros((16,), jnp.float32)
        lane = lax.iota(jnp.int32, 16)
        tail_mask = lane >= 8

        for c in range(NCHUNK):
            base = wid * ROWS_PER_W + c * CH
            pltpu.sync_copy(x_hbm.at[pl.ds(base * L, CH * L)], idx_v)

            ZUNROLL = 32

            def zero_blk(i, carry):
                for u in range(ZUNROLL):
                    hist_v[pl.ds((i * ZUNROLL + u) * 16, 16)] = zeros
                return carry

            lax.fori_loop(0, CH * VPAD // (16 * ZUNROLL), zero_blk, 0)

            def do_row(r, carry):
                rbase = r * VPAD
                for j in range(NVEC):
                    ids = idx_v[pl.ds(r * L + j * 16, 16)]
                    plsc.addupdate_scatter(hist_v, [ids + rbase], ones)
                ids = idx_v[pl.ds(r * L + L - 16, 16)]
                plsc.addupdate_scatter(hist_v, [ids + rbase], ones, mask=tail_mask)
                return carry

            lax.fori_loop(0, CH, do_row, 0)

            pltpu.sync_copy(hist_v, counts_hbm.at[pl.ds(base * VPAD, CH * VPAD)])

    return hist_kernel(x.reshape(B * L)).reshape(B, VPAD)


BB = 256


def _mlp_body(counts_ref, emb_ref, w1_ref, b1_ref, w2_ref, b2_ref, out_ref):
    c = counts_ref[...]
    denom = 200.0 - c[:, 0:1]
    pooled = jnp.dot(c, emb_ref[...], preferred_element_type=jnp.float32) / denom
    h = jnp.dot(pooled, w1_ref[...], preferred_element_type=jnp.float32) + b1_ref[...]
    h = jnp.maximum(h, 0.0)
    out_ref[...] = (
        jnp.dot(h, w2_ref[...], preferred_element_type=jnp.float32) + b2_ref[...]
    )


def _tc_mlp(counts, emb_z, W1, b1, W2, b2):
    return pl.pallas_call(
        _mlp_body,
        grid=(B // BB,),
        in_specs=[
            pl.BlockSpec((BB, VPAD), lambda i: (i, 0)),
            pl.BlockSpec((VPAD, D), lambda i: (0, 0)),
            pl.BlockSpec((D, D), lambda i: (0, 0)),
            pl.BlockSpec((1, D), lambda i: (0, 0)),
            pl.BlockSpec((D, NCLS), lambda i: (0, 0)),
            pl.BlockSpec((1, NCLS), lambda i: (0, 0)),
        ],
        out_specs=pl.BlockSpec((BB, NCLS), lambda i: (i, 0)),
        out_shape=jax.ShapeDtypeStruct((B, NCLS), jnp.float32),
    )(counts, emb_z, W1, b1.reshape(1, D), W2, b2.reshape(1, NCLS))


def kernel(x, emb, W1, b1, W2, b2):
    counts = _sc_histogram(x.astype(jnp.int32))
    emb_z = jnp.zeros((VPAD, D), emb.dtype).at[1:NCLS].set(emb[1:])
    return _tc_mlp(counts, emb_z, W1, b1, W2, b2)
